# baseline (speedup 1.0000x reference)
import functools

import jax
import jax.numpy as jnp
from jax import lax
from jax.experimental import pallas as pl
from jax.experimental.pallas import tpu as pltpu
from jax.experimental.pallas import tpu_sc as plsc

_B, _L = 64, 512
_SEM, _GRAM, _MORPH, _MODEL = 512, 256, 128, 512
_NL = 10
_N = _B * _L
_FUSE_IN = _MORPH + _MORPH // 4 + _MORPH // 2
_TB = 1024
_K = 4
_PN = _N // _K
_PB = _B // _K


def _make_merged_gather(part):
    info = plsc.get_sparse_core_info()
    nw = info.num_cores * info.num_subcores
    per_w = _PN // nw
    chunk = 64
    n_chunks = per_w // chunk
    mesh = plsc.VectorSubcoreMesh(core_axis_name="c", subcore_axis_name="s")

    @functools.partial(
        pl.kernel,
        mesh=mesh,
        out_type=[
            jax.ShapeDtypeStruct((_PN, _SEM), jnp.float32),
            jax.ShapeDtypeStruct((_PN, 128), jnp.float32),
        ],
        scratch_types=[
            pltpu.VMEM((chunk,), jnp.int32),
            pltpu.VMEM((chunk, _SEM), jnp.float32),
            pltpu.VMEM((chunk, 128), jnp.float32),
            pltpu.SemaphoreType.DMA,
            pltpu.SemaphoreType.DMA,
        ],
    )
    def gather(ids_hbm, emb_hbm, vib_hbm, sem_out, vib_out,
               idx_v, rows_v, vrows_v, s1, s2):
        wid = lax.axis_index("s") * info.num_cores + lax.axis_index("c")
        base0 = wid * per_w

        def body(j, carry):
            obase = base0 + j * chunk
            tok = part * _PN + obase
            row = tok // _L
            col = tok % _L
            pltpu.sync_copy(ids_hbm.at[row, pl.ds(col, chunk)], idx_v)
            g1 = pltpu.async_copy(emb_hbm.at[idx_v], rows_v, s1)
            g2 = pltpu.async_copy(vib_hbm.at[idx_v], vrows_v, s2)
            g1.wait()
            g2.wait()
            pltpu.sync_copy(rows_v, sem_out.at[pl.ds(obase, chunk)])
            pltpu.sync_copy(vrows_v, vib_out.at[pl.ds(obase, chunk)])
            return carry

        lax.fori_loop(0, n_chunks, body, 0)

    return gather


_VOCAB = 50265


def _pad_body(vib_ref, out_ref):
    out_ref[...] = jnp.pad(vib_ref[...], ((0, 0), (0, 128 - _NL)))


_PAD_CALL = pl.pallas_call(
    _pad_body,
    grid=((_VOCAB + 511) // 512,),
    in_specs=[pl.BlockSpec((512, _NL), lambda i: (i, 0))],
    out_specs=pl.BlockSpec((512, 128), lambda i: (i, 0)),
    out_shape=jax.ShapeDtypeStruct((_VOCAB, 128), jnp.float32),
)


def _tc_body(sem_ref, vib_ref, pos_ref, cb_ref, depw_ref, depb_ref,
             ln1g_ref, ln1b_ref, fused_ref, fusew_ref, fuseb_ref,
             lnmg_ref, lnmb_ref, projw_ref, projb_ref, lnpg_ref, lnpb_ref,
             out_ref):
    f32 = jnp.float32
    cdim = (((1,), (1,)), ((), ()))
    rdim = (((1,), (0,)), ((), ()))

    posb = pos_ref[...]
    x = sem_ref[...] + jnp.concatenate([posb] * (_TB // _L), axis=0)
    m = jnp.mean(x, axis=-1, keepdims=True)
    v = jnp.mean((x - m) ** 2, axis=-1, keepdims=True)
    semn = (x - m) * lax.rsqrt(v + 1e-5) * ln1g_ref[...] + ln1b_ref[...]

    ws = projw_ref[:, 0:_SEM]
    wc = projw_ref[:, _SEM:_SEM + _GRAM]
    wm = projw_ref[:, _SEM + _GRAM:]
    acc = lax.dot_general(semn, ws, cdim, preferred_element_type=f32)

    logits = vib_ref[:, 0:16]
    lane = lax.broadcasted_iota(jnp.int32, logits.shape, 1)
    valid = lane < _NL
    lm = jnp.max(jnp.where(valid, logits, jnp.float32(-1e30)), axis=-1,
                 keepdims=True)
    conf = 1.0 / jnp.sum(jnp.where(valid, jnp.exp(logits - lm), 0.0),
                         axis=-1, keepdims=True)
    is_max = jnp.logical_and(valid, logits >= lm)
    case = jnp.min(jnp.where(is_max, lane, jnp.int32(127)), axis=-1,
                   keepdims=True)
    onehot = (lane == case).astype(f32)

    cb = cb_ref[...]
    row8 = (lax.broadcasted_iota(jnp.int32, (16, 1), 0) == _NL - 2).astype(f32)
    a = cb + row8 * lax.dot_general(cb, depw_ref[...], cdim,
                                    preferred_element_type=f32)
    t1 = lax.dot_general(a, wc, cdim, preferred_element_type=f32)
    t2 = row8 * lax.dot_general(depb_ref[...], wc, cdim,
                                preferred_element_type=f32)
    oh2 = jnp.concatenate([onehot * conf, onehot], axis=1)
    t12 = jnp.concatenate([t1, t2], axis=0)
    acc += lax.dot_general(oh2, t12, rdim, preferred_element_type=f32)

    fm = lax.dot_general(fused_ref[...], fusew_ref[...], cdim,
                         preferred_element_type=f32) + fuseb_ref[...]
    mm = jnp.mean(fm, axis=-1, keepdims=True)
    mv = jnp.mean((fm - mm) ** 2, axis=-1, keepdims=True)
    morph = (fm - mm) * lax.rsqrt(mv + 1e-5) * lnmg_ref[...] + lnmb_ref[...]
    cvec = lax.dot_general(morph, wm, cdim,
                           preferred_element_type=f32) + projb_ref[...]
    acc += cvec

    g = acc * 0.5 * (1.0 + lax.erf(acc * (2.0 ** -0.5)))
    gm = jnp.mean(g, axis=-1, keepdims=True)
    gv = jnp.mean((g - gm) ** 2, axis=-1, keepdims=True)
    res = (g - gm) * lax.rsqrt(gv + 1e-5) * lnpg_ref[...] + lnpb_ref[...]
    out_ref[...] = res.reshape(_TB // _L, _L, _MODEL)


def _tc_body_chain(prev_ref, *refs):
    del prev_ref
    _tc_body(*refs)


def _const(shape):
    return pl.BlockSpec(shape, lambda i: (0,) * len(shape))


def _make_tc_call(part):
    data_specs = [
        pl.BlockSpec((_TB, _SEM), lambda i: (i, 0)),
        pl.BlockSpec((_TB, 128), lambda i: (i, 0)),
        _const((_L, _SEM)),
        _const((16, _GRAM)),
        _const((_GRAM, _GRAM)),
        _const((1, _GRAM)),
        _const((1, _SEM)),
        _const((1, _SEM)),
        _const((1, _FUSE_IN)),
        _const((_MORPH, _FUSE_IN)),
        _const((1, _MORPH)),
        _const((1, _MORPH)),
        _const((1, _MORPH)),
        _const((_MODEL, _SEM + _GRAM + _MORPH)),
        _const((1, _MODEL)),
        _const((1, _MODEL)),
        _const((1, _MODEL)),
    ]
    rows_per_blk = _TB // _L
    out_spec = pl.BlockSpec(
        (rows_per_blk, _L, _MODEL),
        lambda i: (part * (_PB // rows_per_blk) + i, 0, 0))
    out_shape = jax.ShapeDtypeStruct((_B, _L, _MODEL), jnp.float32)
    grid = (_PN // _TB,)
    if part == 0:
        return pl.pallas_call(
            _tc_body, grid=grid, in_specs=data_specs,
            out_specs=out_spec, out_shape=out_shape)
    return pl.pallas_call(
        _tc_body_chain, grid=grid,
        in_specs=[pl.BlockSpec(memory_space=pl.ANY)] + data_specs,
        out_specs=out_spec, out_shape=out_shape,
        input_output_aliases={0: 0})


def kernel(input_ids, attention_mask, emb, pos, vib_table, case_basis,
           dep_W, dep_b, conf_scale, ln1_g, ln1_b, sc_emb, gn_emb, tt_emb,
           fuse_W, fuse_b, lnm_g, lnm_b, proj_W, proj_b, lnp_g, lnp_b):
    del attention_mask
    vib128 = _PAD_CALL(vib_table)
    parts = [_make_merged_gather(p)(input_ids, emb, vib128)
             for p in range(_K)]

    cb16 = jnp.pad(case_basis * conf_scale[0], ((0, 6), (0, 0)))
    fused = jnp.concatenate([sc_emb[0], gn_emb[0], tt_emb[5]])[None, :]
    weights = (pos[:_L], cb16, dep_W, dep_b[None, :],
               ln1_g[None, :], ln1_b[None, :], fused, fuse_W,
               fuse_b[None, :], lnm_g[None, :], lnm_b[None, :], proj_W,
               proj_b[None, :], lnp_g[None, :], lnp_b[None, :])
    out = _make_tc_call(0)(parts[0][0], parts[0][1], *weights)
    for p in range(1, _K):
        out = _make_tc_call(p)(out, parts[p][0], parts[p][1], *weights)
    return out


# speedup vs baseline: 1.1843x; 1.1843x over previous
import functools

import jax
import jax.numpy as jnp
from jax import lax
from jax.experimental import pallas as pl
from jax.experimental.pallas import tpu as pltpu
from jax.experimental.pallas import tpu_sc as plsc

_B, _L = 64, 512
_SEM, _GRAM, _MORPH, _MODEL = 512, 256, 128, 512
_NL = 10
_N = _B * _L
_FUSE_IN = _MORPH + _MORPH // 4 + _MORPH // 2
_TB = 1024
_K = 4
_PN = _N // _K
_PB = _B // _K


def _make_merged_gather(part):
    info = plsc.get_sparse_core_info()
    nw = info.num_cores * info.num_subcores
    per_w = _PN // nw
    chunk = 64
    n_chunks = per_w // chunk
    mesh = plsc.VectorSubcoreMesh(core_axis_name="c", subcore_axis_name="s")

    @functools.partial(
        pl.kernel,
        mesh=mesh,
        out_type=[
            jax.ShapeDtypeStruct((_PN, _SEM), jnp.float32),
            jax.ShapeDtypeStruct((_PN, 128), jnp.float32),
        ],
        scratch_types=[
            pltpu.VMEM((chunk,), jnp.int32),
            pltpu.VMEM((chunk, _SEM), jnp.float32),
            pltpu.VMEM((chunk, 128), jnp.float32),
            pltpu.SemaphoreType.DMA,
            pltpu.SemaphoreType.DMA,
        ],
    )
    def gather(ids_hbm, emb_hbm, vib_hbm, sem_out, vib_out,
               idx_v, rows_v, vrows_v, s1, s2):
        wid = lax.axis_index("s") * info.num_cores + lax.axis_index("c")
        base0 = wid * per_w

        def body(j, carry):
            obase = base0 + j * chunk
            tok = part * _PN + obase
            row = tok // _L
            col = tok % _L
            pltpu.sync_copy(ids_hbm.at[row, pl.ds(col, chunk)], idx_v)
            g1 = pltpu.async_copy(emb_hbm.at[idx_v], rows_v, s1)
            g2 = pltpu.async_copy(vib_hbm.at[idx_v], vrows_v, s2)
            g1.wait()
            g2.wait()
            pltpu.sync_copy(rows_v, sem_out.at[pl.ds(obase, chunk)])
            pltpu.sync_copy(vrows_v, vib_out.at[pl.ds(obase, chunk)])
            return carry

        lax.fori_loop(0, n_chunks, body, 0)

    return gather


def _make_row_gather(part, width):
    info = plsc.get_sparse_core_info()
    nw = info.num_cores * info.num_subcores
    per_w = _PN // nw
    chunk = 64
    n_chunks = per_w // chunk
    mesh = plsc.VectorSubcoreMesh(core_axis_name="c", subcore_axis_name="s")

    @functools.partial(
        pl.kernel,
        mesh=mesh,
        out_type=jax.ShapeDtypeStruct((_PN, width), jnp.float32),
        scratch_types=[
            pltpu.VMEM((chunk,), jnp.int32),
            pltpu.VMEM((chunk, width), jnp.float32),
            pltpu.SemaphoreType.DMA,
        ],
    )
    def gather(ids_hbm, table_hbm, out, idx_v, rows_v, s1):
        wid = lax.axis_index("s") * info.num_cores + lax.axis_index("c")
        base0 = wid * per_w

        def body(j, carry):
            obase = base0 + j * chunk
            tok = part * _PN + obase
            row = tok // _L
            col = tok % _L
            pltpu.sync_copy(ids_hbm.at[row, pl.ds(col, chunk)], idx_v)
            pltpu.async_copy(table_hbm.at[idx_v], rows_v, s1).wait()
            pltpu.sync_copy(rows_v, out.at[pl.ds(obase, chunk)])
            return carry

        lax.fori_loop(0, n_chunks, body, 0)

    return gather


def _tc_body(sem_ref, vib_ref, pos_ref, cb_ref, depw_ref, depb_ref,
             ln1g_ref, ln1b_ref, fused_ref, fusew_ref, fuseb_ref,
             lnmg_ref, lnmb_ref, projw_ref, projb_ref, lnpg_ref, lnpb_ref,
             out_ref):
    f32 = jnp.float32
    cdim = (((1,), (1,)), ((), ()))
    rdim = (((1,), (0,)), ((), ()))

    posb = pos_ref[...]
    x = sem_ref[...] + jnp.concatenate([posb] * (_TB // _L), axis=0)
    m = jnp.mean(x, axis=-1, keepdims=True)
    v = jnp.mean((x - m) ** 2, axis=-1, keepdims=True)
    semn = (x - m) * lax.rsqrt(v + 1e-5) * ln1g_ref[...] + ln1b_ref[...]

    ws = projw_ref[:, 0:_SEM]
    wc = projw_ref[:, _SEM:_SEM + _GRAM]
    wm = projw_ref[:, _SEM + _GRAM:]
    acc = lax.dot_general(semn, ws, cdim, preferred_element_type=f32)

    logits = vib_ref[:, 0:16]
    lane = lax.broadcasted_iota(jnp.int32, logits.shape, 1)
    valid = lane < _NL
    lm = jnp.max(jnp.where(valid, logits, jnp.float32(-1e30)), axis=-1,
                 keepdims=True)
    conf = 1.0 / jnp.sum(jnp.where(valid, jnp.exp(logits - lm), 0.0),
                         axis=-1, keepdims=True)
    is_max = jnp.logical_and(valid, logits >= lm)
    case = jnp.min(jnp.where(is_max, lane, jnp.int32(127)), axis=-1,
                   keepdims=True)
    onehot = (lane == case).astype(f32)

    cb = cb_ref[...]
    row8 = (lax.broadcasted_iota(jnp.int32, (16, 1), 0) == _NL - 2).astype(f32)
    a = cb + row8 * lax.dot_general(cb, depw_ref[...], cdim,
                                    preferred_element_type=f32)
    t1 = lax.dot_general(a, wc, cdim, preferred_element_type=f32)
    t2 = row8 * lax.dot_general(depb_ref[...], wc, cdim,
                                preferred_element_type=f32)
    oh2 = jnp.concatenate([onehot * conf, onehot], axis=1)
    t12 = jnp.concatenate([t1, t2], axis=0)
    acc += lax.dot_general(oh2, t12, rdim, preferred_element_type=f32)

    fm = lax.dot_general(fused_ref[...], fusew_ref[...], cdim,
                         preferred_element_type=f32) + fuseb_ref[...]
    mm = jnp.mean(fm, axis=-1, keepdims=True)
    mv = jnp.mean((fm - mm) ** 2, axis=-1, keepdims=True)
    morph = (fm - mm) * lax.rsqrt(mv + 1e-5) * lnmg_ref[...] + lnmb_ref[...]
    cvec = lax.dot_general(morph, wm, cdim,
                           preferred_element_type=f32) + projb_ref[...]
    acc += cvec

    g = acc * 0.5 * (1.0 + lax.erf(acc * (2.0 ** -0.5)))
    gm = jnp.mean(g, axis=-1, keepdims=True)
    gv = jnp.mean((g - gm) ** 2, axis=-1, keepdims=True)
    res = (g - gm) * lax.rsqrt(gv + 1e-5) * lnpg_ref[...] + lnpb_ref[...]
    out_ref[...] = res.reshape(_TB // _L, _L, _MODEL)


def _tc_body_chain(prev_ref, *refs):
    del prev_ref
    _tc_body(*refs)


def _const(shape):
    return pl.BlockSpec(shape, lambda i: (0,) * len(shape))


def _make_tc_call(part):
    data_specs = [
        pl.BlockSpec((_TB, _SEM), lambda i: (i, 0)),
        pl.BlockSpec((_TB, 128), lambda i: (i, 0)),
        _const((_L, _SEM)),
        _const((16, _GRAM)),
        _const((_GRAM, _GRAM)),
        _const((1, _GRAM)),
        _const((1, _SEM)),
        _const((1, _SEM)),
        _const((1, _FUSE_IN)),
        _const((_MORPH, _FUSE_IN)),
        _const((1, _MORPH)),
        _const((1, _MORPH)),
        _const((1, _MORPH)),
        _const((_MODEL, _SEM + _GRAM + _MORPH)),
        _const((1, _MODEL)),
        _const((1, _MODEL)),
        _const((1, _MODEL)),
    ]
    rows_per_blk = _TB // _L
    out_spec = pl.BlockSpec(
        (rows_per_blk, _L, _MODEL),
        lambda i: (part * (_PB // rows_per_blk) + i, 0, 0))
    out_shape = jax.ShapeDtypeStruct((_B, _L, _MODEL), jnp.float32)
    grid = (_PN // _TB,)
    if part == 0:
        return pl.pallas_call(
            _tc_body, grid=grid, in_specs=data_specs,
            out_specs=out_spec, out_shape=out_shape)
    return pl.pallas_call(
        _tc_body_chain, grid=grid,
        in_specs=[pl.BlockSpec(memory_space=pl.ANY)] + data_specs,
        out_specs=out_spec, out_shape=out_shape,
        input_output_aliases={0: 0})


def kernel(input_ids, attention_mask, emb, pos, vib_table, case_basis,
           dep_W, dep_b, conf_scale, ln1_g, ln1_b, sc_emb, gn_emb, tt_emb,
           fuse_W, fuse_b, lnm_g, lnm_b, proj_W, proj_b, lnp_g, lnp_b):
    del attention_mask
    vib128 = jnp.pad(vib_table, ((0, 0), (0, 128 - _NL)))
    sem_0 = _make_row_gather(0, _SEM)(input_ids, emb)
    after_emb0 = (sem_0[0, 0] * 0.0).astype(jnp.int32)
    vib_0 = _make_row_gather(0, 128)(input_ids + after_emb0, vib128)
    parts = [(sem_0, vib_0)] + [
        _make_merged_gather(p)(input_ids, emb, vib128)
        for p in range(1, _K)]

    cb16 = jnp.pad(case_basis * conf_scale[0], ((0, 6), (0, 0)))
    fused = jnp.concatenate([sc_emb[0], gn_emb[0], tt_emb[5]])[None, :]
    weights = (pos[:_L], cb16, dep_W, dep_b[None, :],
               ln1_g[None, :], ln1_b[None, :], fused, fuse_W,
               fuse_b[None, :], lnm_g[None, :], lnm_b[None, :], proj_W,
               proj_b[None, :], lnp_g[None, :], lnp_b[None, :])
    out = _make_tc_call(0)(parts[0][0], parts[0][1], *weights)
    for p in range(1, _K):
        out = _make_tc_call(p)(out, parts[p][0], parts[p][1], *weights)
    return out


# speedup vs baseline: 1.2256x; 1.0349x over previous
import functools

import jax
import jax.numpy as jnp
from jax import lax
from jax.experimental import pallas as pl
from jax.experimental.pallas import tpu as pltpu
from jax.experimental.pallas import tpu_sc as plsc

_B, _L = 64, 512
_SEM, _GRAM, _MORPH, _MODEL = 512, 256, 128, 512
_NL = 10
_N = _B * _L
_FUSE_IN = _MORPH + _MORPH // 4 + _MORPH // 2
_TB = 1024
_K = 4
_PN = _N // _K
_PB = _B // _K


def _make_merged_gather(part):
    info = plsc.get_sparse_core_info()
    nw = info.num_cores * info.num_subcores
    per_w = _PN // nw
    chunk = 64
    n_chunks = per_w // chunk
    mesh = plsc.VectorSubcoreMesh(core_axis_name="c", subcore_axis_name="s")

    @functools.partial(
        pl.kernel,
        mesh=mesh,
        out_type=[
            jax.ShapeDtypeStruct((_PN, _SEM), jnp.float32),
            jax.ShapeDtypeStruct((_PN, 128), jnp.float32),
        ],
        scratch_types=[
            pltpu.VMEM((chunk,), jnp.int32),
            pltpu.VMEM((chunk, _SEM), jnp.float32),
            pltpu.VMEM((chunk, 128), jnp.float32),
            pltpu.SemaphoreType.DMA,
            pltpu.SemaphoreType.DMA,
        ],
    )
    def gather(ids_hbm, emb_hbm, vib_hbm, sem_out, vib_out,
               idx_v, rows_v, vrows_v, s1, s2):
        wid = lax.axis_index("s") * info.num_cores + lax.axis_index("c")
        base0 = wid * per_w

        def body(j, carry):
            obase = base0 + j * chunk
            tok = part * _PN + obase
            row = tok // _L
            col = tok % _L
            pltpu.sync_copy(ids_hbm.at[row, pl.ds(col, chunk)], idx_v)
            g1 = pltpu.async_copy(emb_hbm.at[idx_v], rows_v, s1)
            g2 = pltpu.async_copy(vib_hbm.at[idx_v], vrows_v, s2)
            g1.wait()
            g2.wait()
            pltpu.sync_copy(rows_v, sem_out.at[pl.ds(obase, chunk)])
            pltpu.sync_copy(vrows_v, vib_out.at[pl.ds(obase, chunk)])
            return carry

        lax.fori_loop(0, n_chunks, body, 0)

    return gather


def _make_row_gather(part, width):
    info = plsc.get_sparse_core_info()
    nw = info.num_cores * info.num_subcores
    per_w = _PN // nw
    chunk = 64
    n_chunks = per_w // chunk
    mesh = plsc.VectorSubcoreMesh(core_axis_name="c", subcore_axis_name="s")

    @functools.partial(
        pl.kernel,
        mesh=mesh,
        out_type=jax.ShapeDtypeStruct((_PN, width), jnp.float32),
        scratch_types=[
            pltpu.VMEM((chunk,), jnp.int32),
            pltpu.VMEM((chunk, width), jnp.float32),
            pltpu.SemaphoreType.DMA,
        ],
    )
    def gather(ids_hbm, table_hbm, out, idx_v, rows_v, s1):
        wid = lax.axis_index("s") * info.num_cores + lax.axis_index("c")
        base0 = wid * per_w

        def body(j, carry):
            obase = base0 + j * chunk
            tok = part * _PN + obase
            row = tok // _L
            col = tok % _L
            pltpu.sync_copy(ids_hbm.at[row, pl.ds(col, chunk)], idx_v)
            pltpu.async_copy(table_hbm.at[idx_v], rows_v, s1).wait()
            pltpu.sync_copy(rows_v, out.at[pl.ds(obase, chunk)])
            return carry

        lax.fori_loop(0, n_chunks, body, 0)

    return gather


def _tc_body(sem_ref, vib_ref, pos_ref, cb_ref, depw_ref, depb_ref,
             ln1g_ref, ln1b_ref, fused_ref, fusew_ref, fuseb_ref,
             lnmg_ref, lnmb_ref, projw_ref, projb_ref, lnpg_ref, lnpb_ref,
             out_ref):
    f32 = jnp.float32
    cdim = (((1,), (1,)), ((), ()))
    rdim = (((1,), (0,)), ((), ()))

    ones = jnp.full((_SEM, 1), 1.0, f32)
    posb = pos_ref[...]
    x = sem_ref[...] + jnp.concatenate([posb] * (_TB // _L), axis=0)
    m = lax.dot_general(x, ones, rdim, preferred_element_type=f32) * (1.0 / _SEM)
    ssq = lax.dot_general(x * x, ones, rdim,
                          preferred_element_type=f32) * (1.0 / _SEM)
    v = ssq - m * m
    semn = (x - m) * lax.rsqrt(v + 1e-5) * ln1g_ref[...] + ln1b_ref[...]

    ws = projw_ref[:, 0:_SEM]
    wc = projw_ref[:, _SEM:_SEM + _GRAM]
    wm = projw_ref[:, _SEM + _GRAM:]
    acc = lax.dot_general(semn, ws, cdim, preferred_element_type=f32)

    logits = vib_ref[:, 0:16]
    lane = lax.broadcasted_iota(jnp.int32, logits.shape, 1)
    valid = lane < _NL
    lm = jnp.max(jnp.where(valid, logits, jnp.float32(-1e30)), axis=-1,
                 keepdims=True)
    conf = 1.0 / jnp.sum(jnp.where(valid, jnp.exp(logits - lm), 0.0),
                         axis=-1, keepdims=True)
    is_max = jnp.logical_and(valid, logits >= lm)
    case = jnp.min(jnp.where(is_max, lane, jnp.int32(127)), axis=-1,
                   keepdims=True)
    onehot = (lane == case).astype(f32)

    cb = cb_ref[...]
    row8 = (lax.broadcasted_iota(jnp.int32, (16, 1), 0) == _NL - 2).astype(f32)
    a = cb + row8 * lax.dot_general(cb, depw_ref[...], cdim,
                                    preferred_element_type=f32)
    t1 = lax.dot_general(a, wc, cdim, preferred_element_type=f32)
    t2 = row8 * lax.dot_general(depb_ref[...], wc, cdim,
                                preferred_element_type=f32)
    oh2 = jnp.concatenate([onehot * conf, onehot], axis=1)
    t12 = jnp.concatenate([t1, t2], axis=0)
    acc += lax.dot_general(oh2, t12, rdim, preferred_element_type=f32)

    fm = lax.dot_general(fused_ref[...], fusew_ref[...], cdim,
                         preferred_element_type=f32) + fuseb_ref[...]
    mm = jnp.mean(fm, axis=-1, keepdims=True)
    mv = jnp.mean((fm - mm) ** 2, axis=-1, keepdims=True)
    morph = (fm - mm) * lax.rsqrt(mv + 1e-5) * lnmg_ref[...] + lnmb_ref[...]
    cvec = lax.dot_general(morph, wm, cdim,
                           preferred_element_type=f32) + projb_ref[...]
    acc += cvec

    g = acc * 0.5 * (1.0 + lax.erf(acc * (2.0 ** -0.5)))
    gm = lax.dot_general(g, ones, rdim, preferred_element_type=f32) * (1.0 / _MODEL)
    gsq = lax.dot_general(g * g, ones, rdim,
                          preferred_element_type=f32) * (1.0 / _MODEL)
    gv = gsq - gm * gm
    res = (g - gm) * lax.rsqrt(gv + 1e-5) * lnpg_ref[...] + lnpb_ref[...]
    out_ref[...] = res.reshape(_TB // _L, _L, _MODEL)


def _tc_body_chain(prev_ref, *refs):
    del prev_ref
    _tc_body(*refs)


def _const(shape):
    return pl.BlockSpec(shape, lambda i: (0,) * len(shape))


def _make_tc_call(part):
    data_specs = [
        pl.BlockSpec((_TB, _SEM), lambda i: (i, 0)),
        pl.BlockSpec((_TB, 128), lambda i: (i, 0)),
        _const((_L, _SEM)),
        _const((16, _GRAM)),
        _const((_GRAM, _GRAM)),
        _const((1, _GRAM)),
        _const((1, _SEM)),
        _const((1, _SEM)),
        _const((1, _FUSE_IN)),
        _const((_MORPH, _FUSE_IN)),
        _const((1, _MORPH)),
        _const((1, _MORPH)),
        _const((1, _MORPH)),
        _const((_MODEL, _SEM + _GRAM + _MORPH)),
        _const((1, _MODEL)),
        _const((1, _MODEL)),
        _const((1, _MODEL)),
    ]
    rows_per_blk = _TB // _L
    out_spec = pl.BlockSpec(
        (rows_per_blk, _L, _MODEL),
        lambda i: (part * (_PB // rows_per_blk) + i, 0, 0))
    out_shape = jax.ShapeDtypeStruct((_B, _L, _MODEL), jnp.float32)
    grid = (_PN // _TB,)
    if part == 0:
        return pl.pallas_call(
            _tc_body, grid=grid, in_specs=data_specs,
            out_specs=out_spec, out_shape=out_shape)
    return pl.pallas_call(
        _tc_body_chain, grid=grid,
        in_specs=[pl.BlockSpec(memory_space=pl.ANY)] + data_specs,
        out_specs=out_spec, out_shape=out_shape,
        input_output_aliases={0: 0})


def kernel(input_ids, attention_mask, emb, pos, vib_table, case_basis,
           dep_W, dep_b, conf_scale, ln1_g, ln1_b, sc_emb, gn_emb, tt_emb,
           fuse_W, fuse_b, lnm_g, lnm_b, proj_W, proj_b, lnp_g, lnp_b):
    del attention_mask
    vib128 = jnp.pad(vib_table, ((0, 0), (0, 128 - _NL)))
    sem_0 = _make_row_gather(0, _SEM)(input_ids, emb)
    after_emb0 = (sem_0[0, 0] * 0.0).astype(jnp.int32)
    vib_0 = _make_row_gather(0, 128)(input_ids + after_emb0, vib128)
    parts = [(sem_0, vib_0)] + [
        _make_merged_gather(p)(input_ids, emb, vib128)
        for p in range(1, _K)]

    cb16 = jnp.pad(case_basis * conf_scale[0], ((0, 6), (0, 0)))
    fused = jnp.concatenate([sc_emb[0], gn_emb[0], tt_emb[5]])[None, :]
    weights = (pos[:_L], cb16, dep_W, dep_b[None, :],
               ln1_g[None, :], ln1_b[None, :], fused, fuse_W,
               fuse_b[None, :], lnm_g[None, :], lnm_b[None, :], proj_W,
               proj_b[None, :], lnp_g[None, :], lnp_b[None, :])
    out = _make_tc_call(0)(parts[0][0], parts[0][1], *weights)
    for p in range(1, _K):
        out = _make_tc_call(p)(out, parts[p][0], parts[p][1], *weights)
    return out
